# Initial kernel scaffold; baseline (speedup 1.0000x reference)
import jax
import jax.numpy as jnp
from jax.experimental import pallas as pl


def kernel(x, pos_edge_index, neg_edge_index, W1_pos_l, W1_pos_r, b1_pos, W1_neg_l, W1_neg_r, b1_neg, W2_pos_l, W2_pos_r, b2_pos, W2_neg_l, W2_neg_r, b2_neg):
    raise NotImplementedError("write your pallas kernel here")


# speedup vs baseline: 6.1150x; 6.1150x over previous
import functools

import jax
import jax.numpy as jnp
from jax import lax
from jax.experimental import pallas as pl
from jax.experimental.pallas import tpu as pltpu
from jax.experimental.pallas import tpu_sc as plsc

N = 50000
D_IN = 64
H = 32
E = 400000

NC = 2
NS = 16
LROW = 128
EROWS = 3200
EPAD = EROWS * LROW
TROWS = EROWS // NS
K = 4
NCHUNK = TROWS // K
KC = 8
NCHUNKC = TROWS // KC
NACC = 51200
ZCH = NACC // LROW // NS
DUMP = 3128

ROWB = 2000
GRID = N // ROWB

_f32 = jnp.float32


def _mesh():
    return plsc.VectorSubcoreMesh(
        core_axis_name="c", subcore_axis_name="s", num_cores=NC, num_subcores=NS
    )


_SC_PARAMS = pltpu.CompilerParams(use_tc_tiling_on_sc=False)


def _zero_acc(sub, zb, acc):
    def body(i, carry):
        r = (sub * ZCH + i) * LROW
        pltpu.sync_copy(zb, acc.at[pl.ds(r, LROW)])
        return carry

    lax.fori_loop(0, ZCH, body, 0)


def _aggregate(sub, table, src_rows, dst_rows, sidx, didx, rows, acc, gsem):
    row0 = sub * TROWS

    def chunk(g, carry):
        base = row0 + g * K
        pltpu.sync_copy(src_rows.at[pl.ds(base, K)], sidx)
        pltpu.sync_copy(dst_rows.at[pl.ds(base, K)], didx)
        descs = [
            pltpu.async_copy(table.at[sidx.at[j]], rows.at[j], gsem)
            for j in range(K)
        ]
        for d in descs:
            d.wait()
        for j in range(K):
            pltpu.sync_copy(rows.at[j], acc.at[didx.at[j]], add=True)
        return carry

    lax.fori_loop(0, NCHUNK, chunk, 0)


def _dump(sub, acc, out):
    r0 = pl.multiple_of(sub * DUMP, 8)
    tail = N - (NS - 1) * DUMP

    @pl.when(sub < NS - 1)
    def _():
        pltpu.sync_copy(acc.at[pl.ds(r0, DUMP)], out.at[pl.ds(r0, DUMP)])

    @pl.when(sub == NS - 1)
    def _():
        pltpu.sync_copy(acc.at[pl.ds((NS - 1) * DUMP, tail)],
                        out.at[pl.ds((NS - 1) * DUMP, tail)])


def _sc_counts(pdst, ndst, z8, o8):
    @functools.partial(
        pl.kernel,
        out_type=[
            jax.ShapeDtypeStruct((N, 8), _f32),
            jax.ShapeDtypeStruct((N, 8), _f32),
        ],
        mesh=_mesh(),
        scratch_types=[
            pltpu.VMEM_SHARED((NACC, 8), _f32),
            pltpu.VMEM((KC, LROW), jnp.int32),
            pltpu.VMEM((LROW, 8), _f32),
            pltpu.VMEM((LROW, 8), _f32),
        ],
        compiler_params=_SC_PARAMS,
    )
    def k(pdst_h, ndst_h, z8_h, o8_h, cpos_h, cneg_h,
          cnt, didx, zb8, ob8):
        core = lax.axis_index("c")
        sub = lax.axis_index("s")

        pltpu.sync_copy(z8_h, zb8)
        pltpu.sync_copy(o8_h, ob8)
        _zero_acc(sub, zb8, cnt)
        plsc.subcore_barrier()

        def count(dst_rows):
            row0 = sub * TROWS

            def chunk(g, carry):
                base = row0 + g * KC
                pltpu.sync_copy(dst_rows.at[pl.ds(base, KC)], didx)
                for j in range(KC):
                    pltpu.sync_copy(ob8, cnt.at[didx.at[j]], add=True)
                return carry

            lax.fori_loop(0, NCHUNKC, chunk, 0)

        @pl.when(core == 0)
        def _():
            count(pdst_h)

        @pl.when(core == 1)
        def _():
            count(ndst_h)

        plsc.subcore_barrier()

        @pl.when(core == 0)
        def _():
            _dump(sub, cnt, cpos_h)

        @pl.when(core == 1)
        def _():
            _dump(sub, cnt, cneg_h)

    return k(pdst, ndst, z8, o8)


def _sc_layer1(xp, xn, psrc, pdst, nsrc, ndst, z32):
    @functools.partial(
        pl.kernel,
        out_type=[
            jax.ShapeDtypeStruct((N, H), _f32),
            jax.ShapeDtypeStruct((N, H), _f32),
        ],
        mesh=_mesh(),
        scratch_types=[
            pltpu.VMEM_SHARED((NACC, H), _f32),
            pltpu.VMEM((K, LROW), jnp.int32),
            pltpu.VMEM((K, LROW), jnp.int32),
            pltpu.VMEM((K, LROW, H), _f32),
            pltpu.VMEM((LROW, H), _f32),
            pltpu.SemaphoreType.DMA,
        ],
        compiler_params=_SC_PARAMS,
    )
    def k(xp_h, xn_h, psrc_h, pdst_h, nsrc_h, ndst_h, z32_h,
          spos_h, sneg_h,
          acc, sidx, didx, rows, zb32, gsem):
        core = lax.axis_index("c")
        sub = lax.axis_index("s")

        pltpu.sync_copy(z32_h, zb32)
        _zero_acc(sub, zb32, acc)
        plsc.subcore_barrier()

        @pl.when(core == 0)
        def _():
            _aggregate(sub, xp_h, psrc_h, pdst_h, sidx, didx, rows, acc, gsem)

        @pl.when(core == 1)
        def _():
            _aggregate(sub, xn_h, nsrc_h, ndst_h, sidx, didx, rows, acc, gsem)

        plsc.subcore_barrier()

        @pl.when(core == 0)
        def _():
            _dump(sub, acc, spos_h)

        @pl.when(core == 1)
        def _():
            _dump(sub, acc, sneg_h)

    return k(xp, xn, psrc, pdst, nsrc, ndst, z32)


def _sc_layer2(zp, zn, psrc, pdst, nsrc, ndst, z32):
    @functools.partial(
        pl.kernel,
        out_type=[
            jax.ShapeDtypeStruct((N, H), _f32),
            jax.ShapeDtypeStruct((N, H), _f32),
            jax.ShapeDtypeStruct((N, H), _f32),
            jax.ShapeDtypeStruct((N, H), _f32),
        ],
        mesh=_mesh(),
        scratch_types=[
            pltpu.VMEM_SHARED((NACC, H), _f32),
            pltpu.VMEM((K, LROW), jnp.int32),
            pltpu.VMEM((K, LROW), jnp.int32),
            pltpu.VMEM((K, LROW, H), _f32),
            pltpu.VMEM((LROW, H), _f32),
            pltpu.SemaphoreType.DMA,
        ],
        compiler_params=_SC_PARAMS,
    )
    def k(zp_h, zn_h, psrc_h, pdst_h, nsrc_h, ndst_h, z32_h,
          spp_h, spn_h, snn_h, snp_h,
          acc, sidx, didx, rows, zb32, gsem):
        core = lax.axis_index("c")
        sub = lax.axis_index("s")

        pltpu.sync_copy(z32_h, zb32)
        _zero_acc(sub, zb32, acc)
        plsc.subcore_barrier()

        @pl.when(core == 0)
        def _():
            _aggregate(sub, zp_h, psrc_h, pdst_h, sidx, didx, rows, acc, gsem)

        @pl.when(core == 1)
        def _():
            _aggregate(sub, zn_h, nsrc_h, ndst_h, sidx, didx, rows, acc, gsem)

        plsc.subcore_barrier()

        @pl.when(core == 0)
        def _():
            _dump(sub, acc, spp_h)

        @pl.when(core == 1)
        def _():
            _dump(sub, acc, snn_h)

        plsc.subcore_barrier()
        _zero_acc(sub, zb32, acc)
        plsc.subcore_barrier()

        @pl.when(core == 0)
        def _():
            _aggregate(sub, zn_h, psrc_h, pdst_h, sidx, didx, rows, acc, gsem)

        @pl.when(core == 1)
        def _():
            _aggregate(sub, zp_h, nsrc_h, ndst_h, sidx, didx, rows, acc, gsem)

        plsc.subcore_barrier()

        @pl.when(core == 0)
        def _():
            _dump(sub, acc, spn_h)

        @pl.when(core == 1)
        def _():
            _dump(sub, acc, snp_h)

    return k(zp, zn, psrc, pdst, nsrc, ndst, z32)


def _tc1_body(x_ref, w_ref, xp_ref, xrp_ref, xn_ref, xrn_ref):
    h = jnp.dot(x_ref[...], w_ref[...], preferred_element_type=_f32)
    xp_ref[...] = h[:, 0 * H:1 * H]
    xrp_ref[...] = h[:, 1 * H:2 * H]
    xn_ref[...] = h[:, 2 * H:3 * H]
    xrn_ref[...] = h[:, 3 * H:4 * H]


def _tc1(x, w1):
    rb = lambda w: pl.BlockSpec((ROWB, w), lambda i: (i, 0))
    return pl.pallas_call(
        _tc1_body,
        grid=(GRID,),
        in_specs=[rb(D_IN), pl.BlockSpec((D_IN, 4 * H), lambda i: (0, 0))],
        out_specs=[rb(H)] * 4,
        out_shape=[jax.ShapeDtypeStruct((N, H), _f32)] * 4,
    )(x, w1)


def _tc2_body(sp_ref, cp_ref, xrp_ref, bp_ref, sn_ref, cn_ref, xrn_ref, bn_ref,
              zp_ref, zn_ref):
    invp = 1.0 / jnp.maximum(cp_ref[...][:, 0:1], 1.0)
    invn = 1.0 / jnp.maximum(cn_ref[...][:, 0:1], 1.0)
    zp_ref[...] = jnp.tanh(sp_ref[...] * invp + xrp_ref[...] + bp_ref[...])
    zn_ref[...] = jnp.tanh(sn_ref[...] * invn + xrn_ref[...] + bn_ref[...])


def _tc2(s_pos, c_pos, xrp, b1p, s_neg, c_neg, xrn, b1n):
    rb = lambda w: pl.BlockSpec((ROWB, w), lambda i: (i, 0))
    bb = pl.BlockSpec((1, H), lambda i: (0, 0))
    return pl.pallas_call(
        _tc2_body,
        grid=(GRID,),
        in_specs=[rb(H), rb(8), rb(H), bb, rb(H), rb(8), rb(H), bb],
        out_specs=[rb(H)] * 2,
        out_shape=[jax.ShapeDtypeStruct((N, H), _f32)] * 2,
    )(s_pos, c_pos, xrp, b1p, s_neg, c_neg, xrn, b1n)


def _tc3_body(spp_ref, snn_ref, spn_ref, snp_ref, cp_ref, cn_ref,
              zp_ref, zn_ref, w2p_ref, w2n_ref, bp_ref, bn_ref, out_ref):
    invp = 1.0 / jnp.maximum(cp_ref[...][:, 0:1], 1.0)
    invn = 1.0 / jnp.maximum(cn_ref[...][:, 0:1], 1.0)
    P = jnp.concatenate(
        [spp_ref[...] * invp, snn_ref[...] * invn, zp_ref[...]], axis=1)
    Q = jnp.concatenate(
        [spn_ref[...] * invp, snp_ref[...] * invn, zn_ref[...]], axis=1)
    op = jnp.dot(P, w2p_ref[...], preferred_element_type=_f32) + bp_ref[...]
    on = jnp.dot(Q, w2n_ref[...], preferred_element_type=_f32) + bn_ref[...]
    out_ref[...] = jnp.tanh(jnp.concatenate([op, on], axis=1))


def _tc3(spp, snn, spn, snp, c_pos, c_neg, zp, zn, w2p, w2n, b2p, b2n):
    rb = lambda w: pl.BlockSpec((ROWB, w), lambda i: (i, 0))
    wb = pl.BlockSpec((3 * H, H), lambda i: (0, 0))
    bb = pl.BlockSpec((1, H), lambda i: (0, 0))
    return pl.pallas_call(
        _tc3_body,
        grid=(GRID,),
        in_specs=[rb(H), rb(H), rb(H), rb(H), rb(8), rb(8), rb(H), rb(H),
                  wb, wb, bb, bb],
        out_specs=rb(2 * H),
        out_shape=jax.ShapeDtypeStruct((N, 2 * H), _f32),
    )(spp, snn, spn, snp, c_pos, c_neg, zp, zn, w2p, w2n, b2p, b2n)


def _pad_edges(edge_index):
    src = jnp.concatenate(
        [edge_index[0], jnp.zeros((EPAD - E,), jnp.int32)]).reshape(EROWS, LROW)
    dst = jnp.concatenate(
        [edge_index[1], jnp.full((EPAD - E,), N, jnp.int32)]).reshape(EROWS, LROW)
    return src, dst


def kernel(x, pos_edge_index, neg_edge_index,
           W1_pos_l, W1_pos_r, b1_pos, W1_neg_l, W1_neg_r, b1_neg,
           W2_pos_l, W2_pos_r, b2_pos, W2_neg_l, W2_neg_r, b2_neg):
    psrc, pdst = _pad_edges(pos_edge_index)
    nsrc, ndst = _pad_edges(neg_edge_index)
    z32 = jnp.zeros((LROW, H), _f32)
    z8 = jnp.zeros((LROW, 8), _f32)
    o8 = jnp.ones((LROW, 8), _f32)

    c_pos, c_neg = _sc_counts(pdst, ndst, z8, o8)

    w1 = jnp.concatenate([W1_pos_l, W1_pos_r, W1_neg_l, W1_neg_r], axis=1)
    xp, xrp, xn, xrn = _tc1(x, w1)

    s_pos, s_neg = _sc_layer1(xp, xn, psrc, pdst, nsrc, ndst, z32)

    zp, zn = _tc2(s_pos, c_pos, xrp, b1_pos.reshape(1, H),
                  s_neg, c_neg, xrn, b1_neg.reshape(1, H))

    spp, spn, snn, snp = _sc_layer2(zp, zn, psrc, pdst, nsrc, ndst, z32)

    w2p = jnp.concatenate([W2_pos_l[:H], W2_pos_l[H:], W2_pos_r], axis=0)
    w2n = jnp.concatenate([W2_neg_l[:H], W2_neg_l[H:], W2_neg_r], axis=0)
    return _tc3(spp, snn, spn, snp, c_pos, c_neg, zp, zn,
                w2p, w2n, b2_pos.reshape(1, H), b2_neg.reshape(1, H))


# speedup vs baseline: 7.2764x; 1.1899x over previous
import functools

import jax
import jax.numpy as jnp
from jax import lax
from jax.experimental import pallas as pl
from jax.experimental.pallas import tpu as pltpu
from jax.experimental.pallas import tpu_sc as plsc

N = 50000
D_IN = 64
H = 32
E = 400000

NC = 2
NS = 16
LROW = 128
EROWS = 3200
EPAD = EROWS * LROW
TROWS = EROWS // NS
R = 4
NG = TROWS // R
KC = 8
NCHUNKC = TROWS // KC
NACC = 51200
ZCH = NACC // LROW // NS
DUMP = 3128

ROWB = 2000
GRID = N // ROWB

_f32 = jnp.float32


def _mesh():
    return plsc.VectorSubcoreMesh(
        core_axis_name="c", subcore_axis_name="s", num_cores=NC, num_subcores=NS
    )


_SC_PARAMS = pltpu.CompilerParams(use_tc_tiling_on_sc=False)


def _zero_acc(sub, zb, acc):
    def body(i, carry):
        r = (sub * ZCH + i) * LROW
        pltpu.sync_copy(zb, acc.at[pl.ds(r, LROW)])
        return carry

    lax.fori_loop(0, ZCH, body, 0)


def _aggregate(sub, table, src_rows, dst_rows, sidx, didx, rows, acc, gsems):
    row0 = sub * TROWS
    pltpu.sync_copy(src_rows.at[pl.ds(row0, R)], sidx.at[0])
    pltpu.sync_copy(dst_rows.at[pl.ds(row0, R)], didx.at[0])
    for b in range(R):
        pltpu.async_copy(table.at[sidx.at[0, b]], rows.at[b], gsems[b])

    def group_pair(h, carry):
        for cur in (0, 1):
            g = 2 * h + cur
            nxt = 1 - cur

            def prefetch_idx():
                base = row0 + (g + 1) * R
                pltpu.sync_copy(src_rows.at[pl.ds(base, R)], sidx.at[nxt])
                pltpu.sync_copy(dst_rows.at[pl.ds(base, R)], didx.at[nxt])

            if cur == 0:
                prefetch_idx()
            else:
                pl.when(h < NG // 2 - 1)(prefetch_idx)

            for b in range(R):
                pltpu.make_async_copy(
                    table.at[sidx.at[cur, b]], rows.at[b], gsems[b]).wait()
                pltpu.sync_copy(rows.at[b], acc.at[didx.at[cur, b]], add=True)

                def refire(b=b, nxt=nxt):
                    pltpu.async_copy(
                        table.at[sidx.at[nxt, b]], rows.at[b], gsems[b])

                if cur == 0:
                    refire()
                else:
                    pl.when(h < NG // 2 - 1)(refire)
        return carry

    lax.fori_loop(0, NG // 2, group_pair, 0)


def _dump(sub, acc, out):
    r0 = pl.multiple_of(sub * DUMP, 8)
    tail = N - (NS - 1) * DUMP

    @pl.when(sub < NS - 1)
    def _():
        pltpu.sync_copy(acc.at[pl.ds(r0, DUMP)], out.at[pl.ds(r0, DUMP)])

    @pl.when(sub == NS - 1)
    def _():
        pltpu.sync_copy(acc.at[pl.ds((NS - 1) * DUMP, tail)],
                        out.at[pl.ds((NS - 1) * DUMP, tail)])


def _sc_counts(pdst, ndst, z8, o8):
    @functools.partial(
        pl.kernel,
        out_type=[
            jax.ShapeDtypeStruct((N, 8), _f32),
            jax.ShapeDtypeStruct((N, 8), _f32),
        ],
        mesh=_mesh(),
        scratch_types=[
            pltpu.VMEM_SHARED((NACC, 8), _f32),
            pltpu.VMEM((KC, LROW), jnp.int32),
            pltpu.VMEM((LROW, 8), _f32),
            pltpu.VMEM((LROW, 8), _f32),
        ],
        compiler_params=_SC_PARAMS,
    )
    def k(pdst_h, ndst_h, z8_h, o8_h, cpos_h, cneg_h,
          cnt, didx, zb8, ob8):
        core = lax.axis_index("c")
        sub = lax.axis_index("s")

        pltpu.sync_copy(z8_h, zb8)
        pltpu.sync_copy(o8_h, ob8)
        _zero_acc(sub, zb8, cnt)
        plsc.subcore_barrier()

        def count(dst_rows):
            row0 = sub * TROWS

            def chunk(g, carry):
                base = row0 + g * KC
                pltpu.sync_copy(dst_rows.at[pl.ds(base, KC)], didx)
                for j in range(KC):
                    pltpu.sync_copy(ob8, cnt.at[didx.at[j]], add=True)
                return carry

            lax.fori_loop(0, NCHUNKC, chunk, 0)

        @pl.when(core == 0)
        def _():
            count(pdst_h)

        @pl.when(core == 1)
        def _():
            count(ndst_h)

        plsc.subcore_barrier()

        @pl.when(core == 0)
        def _():
            _dump(sub, cnt, cpos_h)

        @pl.when(core == 1)
        def _():
            _dump(sub, cnt, cneg_h)

    return k(pdst, ndst, z8, o8)


def _sc_layer1(xp, xn, psrc, pdst, nsrc, ndst, z32):
    @functools.partial(
        pl.kernel,
        out_type=[
            jax.ShapeDtypeStruct((N, H), _f32),
            jax.ShapeDtypeStruct((N, H), _f32),
        ],
        mesh=_mesh(),
        scratch_types=[
            pltpu.VMEM_SHARED((NACC, H), _f32),
            pltpu.VMEM((2, R, LROW), jnp.int32),
            pltpu.VMEM((2, R, LROW), jnp.int32),
            pltpu.VMEM((R, LROW, H), _f32),
            pltpu.VMEM((LROW, H), _f32),
            pltpu.SemaphoreType.DMA,
            pltpu.SemaphoreType.DMA,
            pltpu.SemaphoreType.DMA,
            pltpu.SemaphoreType.DMA,
        ],
        compiler_params=_SC_PARAMS,
    )
    def k(xp_h, xn_h, psrc_h, pdst_h, nsrc_h, ndst_h, z32_h,
          spos_h, sneg_h,
          acc, sidx, didx, rows, zb32, g0, g1, g2, g3):
        gsems = [g0, g1, g2, g3]
        core = lax.axis_index("c")
        sub = lax.axis_index("s")

        pltpu.sync_copy(z32_h, zb32)
        _zero_acc(sub, zb32, acc)
        plsc.subcore_barrier()

        @pl.when(core == 0)
        def _():
            _aggregate(sub, xp_h, psrc_h, pdst_h, sidx, didx, rows, acc, gsems)

        @pl.when(core == 1)
        def _():
            _aggregate(sub, xn_h, nsrc_h, ndst_h, sidx, didx, rows, acc, gsems)

        plsc.subcore_barrier()

        @pl.when(core == 0)
        def _():
            _dump(sub, acc, spos_h)

        @pl.when(core == 1)
        def _():
            _dump(sub, acc, sneg_h)

    return k(xp, xn, psrc, pdst, nsrc, ndst, z32)


def _sc_layer2(zp, zn, psrc, pdst, nsrc, ndst, z32):
    @functools.partial(
        pl.kernel,
        out_type=[
            jax.ShapeDtypeStruct((N, H), _f32),
            jax.ShapeDtypeStruct((N, H), _f32),
            jax.ShapeDtypeStruct((N, H), _f32),
            jax.ShapeDtypeStruct((N, H), _f32),
        ],
        mesh=_mesh(),
        scratch_types=[
            pltpu.VMEM_SHARED((NACC, H), _f32),
            pltpu.VMEM((2, R, LROW), jnp.int32),
            pltpu.VMEM((2, R, LROW), jnp.int32),
            pltpu.VMEM((R, LROW, H), _f32),
            pltpu.VMEM((LROW, H), _f32),
            pltpu.SemaphoreType.DMA,
            pltpu.SemaphoreType.DMA,
            pltpu.SemaphoreType.DMA,
            pltpu.SemaphoreType.DMA,
        ],
        compiler_params=_SC_PARAMS,
    )
    def k(zp_h, zn_h, psrc_h, pdst_h, nsrc_h, ndst_h, z32_h,
          spp_h, spn_h, snn_h, snp_h,
          acc, sidx, didx, rows, zb32, g0, g1, g2, g3):
        gsems = [g0, g1, g2, g3]
        core = lax.axis_index("c")
        sub = lax.axis_index("s")

        pltpu.sync_copy(z32_h, zb32)
        _zero_acc(sub, zb32, acc)
        plsc.subcore_barrier()

        @pl.when(core == 0)
        def _():
            _aggregate(sub, zp_h, psrc_h, pdst_h, sidx, didx, rows, acc, gsems)

        @pl.when(core == 1)
        def _():
            _aggregate(sub, zn_h, nsrc_h, ndst_h, sidx, didx, rows, acc, gsems)

        plsc.subcore_barrier()

        @pl.when(core == 0)
        def _():
            _dump(sub, acc, spp_h)

        @pl.when(core == 1)
        def _():
            _dump(sub, acc, snn_h)

        plsc.subcore_barrier()
        _zero_acc(sub, zb32, acc)
        plsc.subcore_barrier()

        @pl.when(core == 0)
        def _():
            _aggregate(sub, zn_h, psrc_h, pdst_h, sidx, didx, rows, acc, gsems)

        @pl.when(core == 1)
        def _():
            _aggregate(sub, zp_h, nsrc_h, ndst_h, sidx, didx, rows, acc, gsems)

        plsc.subcore_barrier()

        @pl.when(core == 0)
        def _():
            _dump(sub, acc, spn_h)

        @pl.when(core == 1)
        def _():
            _dump(sub, acc, snp_h)

    return k(zp, zn, psrc, pdst, nsrc, ndst, z32)


def _tc1_body(x_ref, w_ref, xp_ref, xrp_ref, xn_ref, xrn_ref):
    h = jnp.dot(x_ref[...], w_ref[...], preferred_element_type=_f32)
    xp_ref[...] = h[:, 0 * H:1 * H]
    xrp_ref[...] = h[:, 1 * H:2 * H]
    xn_ref[...] = h[:, 2 * H:3 * H]
    xrn_ref[...] = h[:, 3 * H:4 * H]


def _tc1(x, w1):
    rb = lambda w: pl.BlockSpec((ROWB, w), lambda i: (i, 0))
    return pl.pallas_call(
        _tc1_body,
        grid=(GRID,),
        in_specs=[rb(D_IN), pl.BlockSpec((D_IN, 4 * H), lambda i: (0, 0))],
        out_specs=[rb(H)] * 4,
        out_shape=[jax.ShapeDtypeStruct((N, H), _f32)] * 4,
    )(x, w1)


def _tc2_body(sp_ref, cp_ref, xrp_ref, bp_ref, sn_ref, cn_ref, xrn_ref, bn_ref,
              zp_ref, zn_ref):
    invp = 1.0 / jnp.maximum(cp_ref[...][:, 0:1], 1.0)
    invn = 1.0 / jnp.maximum(cn_ref[...][:, 0:1], 1.0)
    zp_ref[...] = jnp.tanh(sp_ref[...] * invp + xrp_ref[...] + bp_ref[...])
    zn_ref[...] = jnp.tanh(sn_ref[...] * invn + xrn_ref[...] + bn_ref[...])


def _tc2(s_pos, c_pos, xrp, b1p, s_neg, c_neg, xrn, b1n):
    rb = lambda w: pl.BlockSpec((ROWB, w), lambda i: (i, 0))
    bb = pl.BlockSpec((1, H), lambda i: (0, 0))
    return pl.pallas_call(
        _tc2_body,
        grid=(GRID,),
        in_specs=[rb(H), rb(8), rb(H), bb, rb(H), rb(8), rb(H), bb],
        out_specs=[rb(H)] * 2,
        out_shape=[jax.ShapeDtypeStruct((N, H), _f32)] * 2,
    )(s_pos, c_pos, xrp, b1p, s_neg, c_neg, xrn, b1n)


def _tc3_body(spp_ref, snn_ref, spn_ref, snp_ref, cp_ref, cn_ref,
              zp_ref, zn_ref, w2p_ref, w2n_ref, bp_ref, bn_ref, out_ref):
    invp = 1.0 / jnp.maximum(cp_ref[...][:, 0:1], 1.0)
    invn = 1.0 / jnp.maximum(cn_ref[...][:, 0:1], 1.0)
    P = jnp.concatenate(
        [spp_ref[...] * invp, snn_ref[...] * invn, zp_ref[...]], axis=1)
    Q = jnp.concatenate(
        [spn_ref[...] * invp, snp_ref[...] * invn, zn_ref[...]], axis=1)
    op = jnp.dot(P, w2p_ref[...], preferred_element_type=_f32) + bp_ref[...]
    on = jnp.dot(Q, w2n_ref[...], preferred_element_type=_f32) + bn_ref[...]
    out_ref[...] = jnp.tanh(jnp.concatenate([op, on], axis=1))


def _tc3(spp, snn, spn, snp, c_pos, c_neg, zp, zn, w2p, w2n, b2p, b2n):
    rb = lambda w: pl.BlockSpec((ROWB, w), lambda i: (i, 0))
    wb = pl.BlockSpec((3 * H, H), lambda i: (0, 0))
    bb = pl.BlockSpec((1, H), lambda i: (0, 0))
    return pl.pallas_call(
        _tc3_body,
        grid=(GRID,),
        in_specs=[rb(H), rb(H), rb(H), rb(H), rb(8), rb(8), rb(H), rb(H),
                  wb, wb, bb, bb],
        out_specs=rb(2 * H),
        out_shape=jax.ShapeDtypeStruct((N, 2 * H), _f32),
    )(spp, snn, spn, snp, c_pos, c_neg, zp, zn, w2p, w2n, b2p, b2n)


def _pad_edges(edge_index):
    src = jnp.concatenate(
        [edge_index[0], jnp.zeros((EPAD - E,), jnp.int32)]).reshape(EROWS, LROW)
    dst = jnp.concatenate(
        [edge_index[1], jnp.full((EPAD - E,), N, jnp.int32)]).reshape(EROWS, LROW)
    return src, dst


def kernel(x, pos_edge_index, neg_edge_index,
           W1_pos_l, W1_pos_r, b1_pos, W1_neg_l, W1_neg_r, b1_neg,
           W2_pos_l, W2_pos_r, b2_pos, W2_neg_l, W2_neg_r, b2_neg):
    psrc, pdst = _pad_edges(pos_edge_index)
    nsrc, ndst = _pad_edges(neg_edge_index)
    z32 = jnp.zeros((LROW, H), _f32)
    z8 = jnp.zeros((LROW, 8), _f32)
    o8 = jnp.ones((LROW, 8), _f32)

    c_pos, c_neg = _sc_counts(pdst, ndst, z8, o8)

    w1 = jnp.concatenate([W1_pos_l, W1_pos_r, W1_neg_l, W1_neg_r], axis=1)
    xp, xrp, xn, xrn = _tc1(x, w1)

    s_pos, s_neg = _sc_layer1(xp, xn, psrc, pdst, nsrc, ndst, z32)

    zp, zn = _tc2(s_pos, c_pos, xrp, b1_pos.reshape(1, H),
                  s_neg, c_neg, xrn, b1_neg.reshape(1, H))

    spp, spn, snn, snp = _sc_layer2(zp, zn, psrc, pdst, nsrc, ndst, z32)

    w2p = jnp.concatenate([W2_pos_l[:H], W2_pos_l[H:], W2_pos_r], axis=0)
    w2n = jnp.concatenate([W2_neg_l[:H], W2_neg_l[H:], W2_neg_r], axis=0)
    return _tc3(spp, snn, spn, snp, c_pos, c_neg, zp, zn,
                w2p, w2n, b2_pos.reshape(1, H), b2_neg.reshape(1, H))


# speedup vs baseline: 7.4680x; 1.0263x over previous
import functools

import jax
import jax.numpy as jnp
from jax import lax
from jax.experimental import pallas as pl
from jax.experimental.pallas import tpu as pltpu
from jax.experimental.pallas import tpu_sc as plsc

N = 50000
D_IN = 64
H = 32
E = 400000

NC = 2
NS = 16
LROW = 128
EROWS = 3200
EPAD = EROWS * LROW
TROWS = EROWS // NS
R = 4
NG = TROWS // R
NACC = 51200
ZCH = NACC // LROW // NS
DUMP = 3128

ROWB = 5000
GRID = N // ROWB

_f32 = jnp.float32


def _mesh():
    return plsc.VectorSubcoreMesh(
        core_axis_name="c", subcore_axis_name="s", num_cores=NC, num_subcores=NS
    )


_SC_PARAMS = pltpu.CompilerParams(use_tc_tiling_on_sc=False)


def _zero_acc(sub, zb, acc, sem):
    def fire(i, carry):
        r = (sub * ZCH + i) * LROW
        pltpu.async_copy(zb, acc.at[pl.ds(r, LROW)], sem)
        return carry

    lax.fori_loop(0, ZCH, fire, 0)

    def drain(i, carry):
        r = (sub * ZCH + i) * LROW
        pltpu.make_async_copy(zb, acc.at[pl.ds(r, LROW)], sem).wait()
        return carry

    lax.fori_loop(0, ZCH, drain, 0)


def _aggregate(sub, table, src_rows, dst_rows, sidx, didx, rows, acc, gsems,
               cnt=None, ones=None):
    row0 = sub * TROWS
    pltpu.sync_copy(src_rows.at[pl.ds(row0, R)], sidx.at[0])
    pltpu.sync_copy(dst_rows.at[pl.ds(row0, R)], didx.at[0])
    for b in range(R):
        pltpu.async_copy(table.at[sidx.at[0, b]], rows.at[b], gsems[b])

    def group_pair(h, carry):
        for cur in (0, 1):
            g = 2 * h + cur
            nxt = 1 - cur

            def prefetch_idx():
                base = row0 + (g + 1) * R
                pltpu.sync_copy(src_rows.at[pl.ds(base, R)], sidx.at[nxt])
                pltpu.sync_copy(dst_rows.at[pl.ds(base, R)], didx.at[nxt])

            if cur == 0:
                prefetch_idx()
            else:
                pl.when(h < NG // 2 - 1)(prefetch_idx)

            for b in range(R):
                pltpu.make_async_copy(
                    table.at[sidx.at[cur, b]], rows.at[b], gsems[b]).wait()
                pltpu.sync_copy(rows.at[b], acc.at[didx.at[cur, b]], add=True)
                if cnt is not None:
                    pltpu.sync_copy(ones, cnt.at[didx.at[cur, b]], add=True)

                def refire(b=b, nxt=nxt):
                    pltpu.async_copy(
                        table.at[sidx.at[nxt, b]], rows.at[b], gsems[b])

                if cur == 0:
                    refire()
                else:
                    pl.when(h < NG // 2 - 1)(refire)
        return carry

    lax.fori_loop(0, NG // 2, group_pair, 0)


def _dump(sub, acc, out):
    r0 = pl.multiple_of(sub * DUMP, 8)
    tail = N - (NS - 1) * DUMP

    @pl.when(sub < NS - 1)
    def _():
        pltpu.sync_copy(acc.at[pl.ds(r0, DUMP)], out.at[pl.ds(r0, DUMP)])

    @pl.when(sub == NS - 1)
    def _():
        pltpu.sync_copy(acc.at[pl.ds((NS - 1) * DUMP, tail)],
                        out.at[pl.ds((NS - 1) * DUMP, tail)])


def _sc_counts(pdst, ndst, z8, o8):
    @functools.partial(
        pl.kernel,
        out_type=[
            jax.ShapeDtypeStruct((N, 8), _f32),
            jax.ShapeDtypeStruct((N, 8), _f32),
        ],
        mesh=_mesh(),
        scratch_types=[
            pltpu.VMEM_SHARED((NACC, 8), _f32),
            pltpu.VMEM((8, LROW), jnp.int32),
            pltpu.VMEM((LROW, 8), _f32),
            pltpu.VMEM((LROW, 8), _f32),
            pltpu.SemaphoreType.DMA,
        ],
        compiler_params=_SC_PARAMS,
    )
    def k(pdst_h, ndst_h, z8_h, o8_h, cpos_h, cneg_h,
          cnt, didx, zb8, ob8, zsem):
        core = lax.axis_index("c")
        sub = lax.axis_index("s")

        pltpu.sync_copy(z8_h, zb8)
        pltpu.sync_copy(o8_h, ob8)
        _zero_acc(sub, zb8, cnt, zsem)
        plsc.subcore_barrier()

        def count(dst_rows):
            row0 = sub * TROWS

            def chunk(g, carry):
                base = row0 + g * 8
                pltpu.sync_copy(dst_rows.at[pl.ds(base, 8)], didx)
                for j in range(8):
                    pltpu.sync_copy(ob8, cnt.at[didx.at[j]], add=True)
                return carry

            lax.fori_loop(0, TROWS // 8, chunk, 0)

        @pl.when(core == 0)
        def _():
            count(pdst_h)

        @pl.when(core == 1)
        def _():
            count(ndst_h)

        plsc.subcore_barrier()

        @pl.when(core == 0)
        def _():
            _dump(sub, cnt, cpos_h)

        @pl.when(core == 1)
        def _():
            _dump(sub, cnt, cneg_h)

    return k(pdst, ndst, z8, o8)


def _sc_layer1(xp, xn, psrc, pdst, nsrc, ndst, z32):
    @functools.partial(
        pl.kernel,
        out_type=[
            jax.ShapeDtypeStruct((N, H), _f32),
            jax.ShapeDtypeStruct((N, H), _f32),
        ],
        mesh=_mesh(),
        scratch_types=[
            pltpu.VMEM_SHARED((NACC, H), _f32),
            pltpu.VMEM((2, R, LROW), jnp.int32),
            pltpu.VMEM((2, R, LROW), jnp.int32),
            pltpu.VMEM((R, LROW, H), _f32),
            pltpu.VMEM((LROW, H), _f32),
            pltpu.SemaphoreType.DMA,
            pltpu.SemaphoreType.DMA,
            pltpu.SemaphoreType.DMA,
            pltpu.SemaphoreType.DMA,
        ],
        compiler_params=_SC_PARAMS,
    )
    def k(xp_h, xn_h, psrc_h, pdst_h, nsrc_h, ndst_h, z32_h,
          spos_h, sneg_h,
          acc, sidx, didx, rows, zb32, g0, g1, g2, g3):
        gsems = [g0, g1, g2, g3]
        core = lax.axis_index("c")
        sub = lax.axis_index("s")

        pltpu.sync_copy(z32_h, zb32)
        _zero_acc(sub, zb32, acc, g0)
        plsc.subcore_barrier()

        @pl.when(core == 0)
        def _():
            _aggregate(sub, xp_h, psrc_h, pdst_h, sidx, didx, rows, acc, gsems)

        @pl.when(core == 1)
        def _():
            _aggregate(sub, xn_h, nsrc_h, ndst_h, sidx, didx, rows, acc, gsems)

        plsc.subcore_barrier()

        @pl.when(core == 0)
        def _():
            _dump(sub, acc, spos_h)

        @pl.when(core == 1)
        def _():
            _dump(sub, acc, sneg_h)

    return k(xp, xn, psrc, pdst, nsrc, ndst, z32)


def _sc_layer2(zp, zn, psrc, pdst, nsrc, ndst, z32):
    @functools.partial(
        pl.kernel,
        out_type=[
            jax.ShapeDtypeStruct((N, H), _f32),
            jax.ShapeDtypeStruct((N, H), _f32),
            jax.ShapeDtypeStruct((N, H), _f32),
            jax.ShapeDtypeStruct((N, H), _f32),
        ],
        mesh=_mesh(),
        scratch_types=[
            pltpu.VMEM_SHARED((NACC, H), _f32),
            pltpu.VMEM((2, R, LROW), jnp.int32),
            pltpu.VMEM((2, R, LROW), jnp.int32),
            pltpu.VMEM((R, LROW, H), _f32),
            pltpu.VMEM((LROW, H), _f32),
            pltpu.SemaphoreType.DMA,
            pltpu.SemaphoreType.DMA,
            pltpu.SemaphoreType.DMA,
            pltpu.SemaphoreType.DMA,
        ],
        compiler_params=_SC_PARAMS,
    )
    def k(zp_h, zn_h, psrc_h, pdst_h, nsrc_h, ndst_h, z32_h,
          spp_h, spn_h, snn_h, snp_h,
          acc, sidx, didx, rows, zb32, g0, g1, g2, g3):
        gsems = [g0, g1, g2, g3]
        core = lax.axis_index("c")
        sub = lax.axis_index("s")

        pltpu.sync_copy(z32_h, zb32)
        _zero_acc(sub, zb32, acc, g0)
        plsc.subcore_barrier()

        @pl.when(core == 0)
        def _():
            _aggregate(sub, zp_h, psrc_h, pdst_h, sidx, didx, rows, acc, gsems)

        @pl.when(core == 1)
        def _():
            _aggregate(sub, zn_h, nsrc_h, ndst_h, sidx, didx, rows, acc, gsems)

        plsc.subcore_barrier()

        @pl.when(core == 0)
        def _():
            _dump(sub, acc, spp_h)

        @pl.when(core == 1)
        def _():
            _dump(sub, acc, snn_h)

        plsc.subcore_barrier()
        _zero_acc(sub, zb32, acc, g0)
        plsc.subcore_barrier()

        @pl.when(core == 0)
        def _():
            _aggregate(sub, zn_h, psrc_h, pdst_h, sidx, didx, rows, acc, gsems)

        @pl.when(core == 1)
        def _():
            _aggregate(sub, zp_h, nsrc_h, ndst_h, sidx, didx, rows, acc, gsems)

        plsc.subcore_barrier()

        @pl.when(core == 0)
        def _():
            _dump(sub, acc, spn_h)

        @pl.when(core == 1)
        def _():
            _dump(sub, acc, snp_h)

    return k(zp, zn, psrc, pdst, nsrc, ndst, z32)


def _tc1_body(x_ref, w_ref, xp_ref, xrp_ref, xn_ref, xrn_ref):
    h = jnp.dot(x_ref[...], w_ref[...], preferred_element_type=_f32)
    xp_ref[...] = h[:, 0 * H:1 * H]
    xrp_ref[...] = h[:, 1 * H:2 * H]
    xn_ref[...] = h[:, 2 * H:3 * H]
    xrn_ref[...] = h[:, 3 * H:4 * H]


def _tc1(x, w1):
    rb = lambda w: pl.BlockSpec((ROWB, w), lambda i: (i, 0))
    return pl.pallas_call(
        _tc1_body,
        grid=(GRID,),
        in_specs=[rb(D_IN), pl.BlockSpec((D_IN, 4 * H), lambda i: (0, 0))],
        out_specs=[rb(H)] * 4,
        out_shape=[jax.ShapeDtypeStruct((N, H), _f32)] * 4,
    )(x, w1)


def _tc2_body(sp_ref, cp_ref, xrp_ref, bp_ref, sn_ref, cn_ref, xrn_ref, bn_ref,
              zp_ref, zn_ref):
    invp = 1.0 / jnp.maximum(cp_ref[...][:, 0:1], 1.0)
    invn = 1.0 / jnp.maximum(cn_ref[...][:, 0:1], 1.0)
    zp_ref[...] = jnp.tanh(sp_ref[...] * invp + xrp_ref[...] + bp_ref[...])
    zn_ref[...] = jnp.tanh(sn_ref[...] * invn + xrn_ref[...] + bn_ref[...])


def _tc2(s_pos, c_pos, xrp, b1p, s_neg, c_neg, xrn, b1n):
    rb = lambda w: pl.BlockSpec((ROWB, w), lambda i: (i, 0))
    bb = pl.BlockSpec((1, H), lambda i: (0, 0))
    return pl.pallas_call(
        _tc2_body,
        grid=(GRID,),
        in_specs=[rb(H), rb(8), rb(H), bb, rb(H), rb(8), rb(H), bb],
        out_specs=[rb(H)] * 2,
        out_shape=[jax.ShapeDtypeStruct((N, H), _f32)] * 2,
    )(s_pos, c_pos, xrp, b1p, s_neg, c_neg, xrn, b1n)


def _tc3_body(spp_ref, snn_ref, spn_ref, snp_ref, cp_ref, cn_ref,
              zp_ref, zn_ref, w2p_ref, w2n_ref, bp_ref, bn_ref, out_ref):
    invp = 1.0 / jnp.maximum(cp_ref[...][:, 0:1], 1.0)
    invn = 1.0 / jnp.maximum(cn_ref[...][:, 0:1], 1.0)
    P = jnp.concatenate(
        [spp_ref[...] * invp, snn_ref[...] * invn, zp_ref[...]], axis=1)
    Q = jnp.concatenate(
        [spn_ref[...] * invp, snp_ref[...] * invn, zn_ref[...]], axis=1)
    op = jnp.dot(P, w2p_ref[...], preferred_element_type=_f32) + bp_ref[...]
    on = jnp.dot(Q, w2n_ref[...], preferred_element_type=_f32) + bn_ref[...]
    out_ref[...] = jnp.tanh(jnp.concatenate([op, on], axis=1))


def _tc3(spp, snn, spn, snp, c_pos, c_neg, zp, zn, w2p, w2n, b2p, b2n):
    rb = lambda w: pl.BlockSpec((ROWB, w), lambda i: (i, 0))
    wb = pl.BlockSpec((3 * H, H), lambda i: (0, 0))
    bb = pl.BlockSpec((1, H), lambda i: (0, 0))
    return pl.pallas_call(
        _tc3_body,
        grid=(GRID,),
        in_specs=[rb(H), rb(H), rb(H), rb(H), rb(8), rb(8), rb(H), rb(H),
                  wb, wb, bb, bb],
        out_specs=rb(2 * H),
        out_shape=jax.ShapeDtypeStruct((N, 2 * H), _f32),
    )(spp, snn, spn, snp, c_pos, c_neg, zp, zn, w2p, w2n, b2p, b2n)


def _pad_edges(edge_index):
    src = jnp.concatenate(
        [edge_index[0], jnp.zeros((EPAD - E,), jnp.int32)]).reshape(EROWS, LROW)
    dst = jnp.concatenate(
        [edge_index[1], jnp.full((EPAD - E,), N, jnp.int32)]).reshape(EROWS, LROW)
    return src, dst


def kernel(x, pos_edge_index, neg_edge_index,
           W1_pos_l, W1_pos_r, b1_pos, W1_neg_l, W1_neg_r, b1_neg,
           W2_pos_l, W2_pos_r, b2_pos, W2_neg_l, W2_neg_r, b2_neg):
    psrc, pdst = _pad_edges(pos_edge_index)
    nsrc, ndst = _pad_edges(neg_edge_index)
    z32 = jnp.zeros((LROW, H), _f32)
    z8 = jnp.zeros((LROW, 8), _f32)
    o8 = jnp.ones((LROW, 8), _f32)

    c_pos, c_neg = _sc_counts(pdst, ndst, z8, o8)

    w1 = jnp.concatenate([W1_pos_l, W1_pos_r, W1_neg_l, W1_neg_r], axis=1)
    xp, xrp, xn, xrn = _tc1(x, w1)

    s_pos, s_neg = _sc_layer1(xp, xn, psrc, pdst, nsrc, ndst, z32)

    zp, zn = _tc2(s_pos, c_pos, xrp, b1_pos.reshape(1, H),
                  s_neg, c_neg, xrn, b1_neg.reshape(1, H))

    spp, spn, snn, snp = _sc_layer2(zp, zn, psrc, pdst, nsrc, ndst, z32)

    w2p = jnp.concatenate([W2_pos_l[:H], W2_pos_l[H:], W2_pos_r], axis=0)
    w2n = jnp.concatenate([W2_neg_l[:H], W2_neg_l[H:], W2_neg_r], axis=0)
    return _tc3(spp, snn, spn, snp, c_pos, c_neg, zp, zn,
                w2p, w2n, b2_pos.reshape(1, H), b2_neg.reshape(1, H))
